# Initial kernel scaffold; baseline (speedup 1.0000x reference)
import functools

import jax
import jax.numpy as jnp
from jax import lax
from jax.experimental import pallas as pl
from jax.experimental.pallas import tpu as pltpu
from jax.experimental.pallas import tpu_sc as plsc

NC = 2
NS = 16
L = 16
K = 80


def _spmm_partials(n_nodes, d, n_edges):
    nw = NC * NS
    epw = n_edges // nw
    nchunk = epw // K
    rpt = n_nodes // NS
    assert epw * nw == n_edges and nchunk * K == epw and rpt * NS == n_nodes
    assert d % L == 0

    mesh = plsc.VectorSubcoreMesh(core_axis_name="c", subcore_axis_name="s")

    @functools.partial(
        pl.kernel,
        out_type=jax.ShapeDtypeStruct((NC, n_nodes, d), jnp.float32),
        mesh=mesh,
        scratch_types=[
            pltpu.VMEM_SHARED((n_nodes, d), jnp.float32),
            pltpu.VMEM((K,), jnp.int32),
            pltpu.VMEM((K,), jnp.int32),
            pltpu.VMEM((K,), jnp.float32),
            pltpu.VMEM((K, d), jnp.float32),
            pltpu.SemaphoreType.DMA,
        ],
    )
    def spmm(emb_hbm, row_hbm, col_hbm, val_hbm, part_hbm,
             acc_sh, col_v, row_v, val_v, gbuf, sem):
        c = lax.axis_index("c")
        s = lax.axis_index("s")
        base = (c * NS + s) * epw

        def zero_body(t, _):
            gbuf[t // (d // L), pl.ds((t % (d // L)) * L, L)] = jnp.zeros(
                (L,), jnp.float32)
            return 0
        lax.fori_loop(0, K * (d // L), zero_body, 0)
        nfull, rem = rpt // K, rpt % K
        for i in range(nfull):
            pltpu.sync_copy(gbuf, acc_sh.at[pl.ds(s * rpt + i * K, K)])
        if rem:
            pltpu.sync_copy(gbuf.at[pl.ds(0, rem)],
                            acc_sh.at[pl.ds(s * rpt + nfull * K, rem)])
        plsc.subcore_barrier()

        def chunk_body(i, _):
            off = base + i * K
            pltpu.sync_copy(col_hbm.at[pl.ds(off, K)], col_v)
            pltpu.sync_copy(row_hbm.at[pl.ds(off, K)], row_v)
            pltpu.sync_copy(val_hbm.at[pl.ds(off, K)], val_v)
            pltpu.async_copy(emb_hbm.at[col_v], gbuf, sem).wait()

            def scale_body(k, _):
                vv = plsc.load_gather(
                    val_v, [jnp.full((L,), 0, jnp.int32) + k])
                for j in range(d // L):
                    gbuf[k, pl.ds(j * L, L)] = gbuf[k, pl.ds(j * L, L)] * vv
                return 0
            lax.fori_loop(0, K, scale_body, 0)

            pltpu.sync_copy(gbuf, acc_sh.at[row_v], add=True)
            return 0
        lax.fori_loop(0, nchunk, chunk_body, 0)

        plsc.subcore_barrier()
        pltpu.sync_copy(acc_sh.at[pl.ds(s * rpt, rpt)],
                        part_hbm.at[c, pl.ds(s * rpt, rpt)])

    return spmm


def _add_block(a_ref, b_ref, o_ref):
    o_ref[...] = a_ref[0] + b_ref[0]


def _combine(partials, num_user, num_item, d, block):
    nbu, nbi = num_user // block, num_item // block
    assert nbu * block == num_user and nbi * block == num_item

    def call(nblk, row0):
        return pl.pallas_call(
            _add_block,
            grid=(nblk,),
            in_specs=[
                pl.BlockSpec((1, block, d), lambda i: (0, row0 + i, 0)),
                pl.BlockSpec((1, block, d), lambda i: (1, row0 + i, 0)),
            ],
            out_specs=pl.BlockSpec((block, d), lambda i: (i, 0)),
            out_shape=jax.ShapeDtypeStruct((nblk * block, d), jnp.float32),
        )(partials, partials)

    return call(nbu, 0), call(nbi, nbu)


def kernel(users_emb, items_emb, graph_row, graph_col, graph_val):
    num_user, d = users_emb.shape
    num_item = items_emb.shape[0]
    n_nodes = num_user + num_item
    n_edges = graph_row.shape[0]

    all_emb = jnp.concatenate([users_emb, items_emb], axis=0)
    spmm = _spmm_partials(n_nodes, d, n_edges)
    partials = spmm(all_emb, graph_row, graph_col, graph_val)
    return _combine(partials, num_user, num_item, d, 1000)


# speedup vs baseline: 4.5278x; 4.5278x over previous
import functools

import jax
import jax.numpy as jnp
from jax import lax
from jax.experimental import pallas as pl
from jax.experimental.pallas import tpu as pltpu
from jax.experimental.pallas import tpu_sc as plsc

NC = 2
NS = 16
L = 16
K = 80


def _lane_broadcast(vec, e):
    dnums = lax.GatherDimensionNumbers(
        offset_dims=(), collapsed_slice_dims=(0,), start_index_map=(0,))
    return lax.gather(vec, jnp.full((L, 1), e, jnp.int32), dnums, (1,),
                      mode=lax.GatherScatterMode.PROMISE_IN_BOUNDS)


def _spmm_partials(n_nodes, d, n_edges):
    nw = NC * NS
    epw = n_edges // nw
    nchunk = epw // K
    n_pad = -(-n_nodes // (NS * 8)) * (NS * 8)
    rpt = n_pad // NS
    assert epw * nw == n_edges and nchunk * K == epw
    assert d % L == 0

    mesh = plsc.VectorSubcoreMesh(core_axis_name="c", subcore_axis_name="s")

    @functools.partial(
        pl.kernel,
        out_type=jax.ShapeDtypeStruct((NC, n_pad, d), jnp.float32),
        mesh=mesh,
        scratch_types=[
            pltpu.VMEM_SHARED((n_pad, d), jnp.float32),
            pltpu.VMEM((K,), jnp.int32),
            pltpu.VMEM((K,), jnp.int32),
            pltpu.VMEM((K,), jnp.float32),
            pltpu.VMEM((K, d), jnp.float32),
            pltpu.SemaphoreType.DMA,
        ],
    )
    def spmm(emb_hbm, row_hbm, col_hbm, val_hbm, part_hbm,
             acc_sh, col_v, row_v, val_v, gbuf, sem):
        c = lax.axis_index("c")
        s = lax.axis_index("s")
        base = (c * NS + s) * epw

        def zero_body(t, _):
            gbuf[t // (d // L), pl.ds((t % (d // L)) * L, L)] = jnp.zeros(
                (L,), jnp.float32)
            return 0
        lax.fori_loop(0, K * (d // L), zero_body, 0)
        nfull, rem = rpt // K, rpt % K
        for i in range(nfull):
            pltpu.sync_copy(gbuf, acc_sh.at[pl.ds(s * rpt + i * K, K)])
        if rem:
            pltpu.sync_copy(gbuf.at[pl.ds(0, rem)],
                            acc_sh.at[pl.ds(s * rpt + nfull * K, rem)])
        plsc.subcore_barrier()

        def chunk_body(i, _):
            off = base + i * K
            pltpu.sync_copy(col_hbm.at[pl.ds(off, K)], col_v)
            pltpu.sync_copy(row_hbm.at[pl.ds(off, K)], row_v)
            pltpu.sync_copy(val_hbm.at[pl.ds(off, K)], val_v)
            pltpu.async_copy(emb_hbm.at[col_v], gbuf, sem).wait()

            def scale_body(g, _):
                k0 = g * L
                vv16 = val_v[pl.ds(k0, L)]
                for e in range(L):
                    vvb = _lane_broadcast(vv16, e)
                    for j in range(d // L):
                        gbuf[k0 + e, pl.ds(j * L, L)] = (
                            gbuf[k0 + e, pl.ds(j * L, L)] * vvb)
                return 0
            lax.fori_loop(0, K // L, scale_body, 0)

            pltpu.sync_copy(gbuf, acc_sh.at[row_v], add=True)
            return 0
        lax.fori_loop(0, nchunk, chunk_body, 0)

        plsc.subcore_barrier()
        pltpu.sync_copy(acc_sh.at[pl.ds(s * rpt, rpt)],
                        part_hbm.at[c, pl.ds(s * rpt, rpt)])

    return spmm


def _add_block(a_ref, b_ref, o_ref):
    o_ref[...] = a_ref[0] + b_ref[0]


def _combine(partials, num_user, num_item, d, block):
    nbu, nbi = num_user // block, num_item // block
    assert nbu * block == num_user and nbi * block == num_item

    def call(nblk, row0):
        return pl.pallas_call(
            _add_block,
            grid=(nblk,),
            in_specs=[
                pl.BlockSpec((1, block, d), lambda i: (0, row0 + i, 0)),
                pl.BlockSpec((1, block, d), lambda i: (1, row0 + i, 0)),
            ],
            out_specs=pl.BlockSpec((block, d), lambda i: (i, 0)),
            out_shape=jax.ShapeDtypeStruct((nblk * block, d), jnp.float32),
        )(partials, partials)

    return call(nbu, 0), call(nbi, nbu)


def kernel(users_emb, items_emb, graph_row, graph_col, graph_val):
    num_user, d = users_emb.shape
    num_item = items_emb.shape[0]
    n_nodes = num_user + num_item
    n_edges = graph_row.shape[0]

    all_emb = jnp.concatenate([users_emb, items_emb], axis=0)
    spmm = _spmm_partials(n_nodes, d, n_edges)
    partials = spmm(all_emb, graph_row, graph_col, graph_val)
    return _combine(partials, num_user, num_item, d, 1000)


# speedup vs baseline: 11.3906x; 2.5157x over previous
import functools

import jax
import jax.numpy as jnp
from jax import lax
from jax.experimental import pallas as pl
from jax.experimental.pallas import tpu as pltpu
from jax.experimental.pallas import tpu_sc as plsc

NC = 2
NS = 16
L = 16
K = 80


def _lane_broadcast(vec, e):
    dnums = lax.GatherDimensionNumbers(
        offset_dims=(), collapsed_slice_dims=(0,), start_index_map=(0,))
    return lax.gather(vec, jnp.full((L, 1), e, jnp.int32), dnums, (1,),
                      mode=lax.GatherScatterMode.PROMISE_IN_BOUNDS)


def _spmm_partials(n_nodes, d, n_edges):
    shift = max(1, (n_nodes - 1).bit_length())
    mask = (1 << shift) - 1
    assert n_nodes * (1 << shift) <= 2**31
    nw = NC * NS
    epw = n_edges // nw
    nchunk = epw // K
    n_pad = -(-n_nodes // (NS * 8)) * (NS * 8)
    rpt = n_pad // NS
    assert epw * nw == n_edges and nchunk * K == epw
    assert d % L == 0 and K % L == 0

    mesh = plsc.VectorSubcoreMesh(core_axis_name="c", subcore_axis_name="s")

    @functools.partial(
        pl.kernel,
        out_type=jax.ShapeDtypeStruct((NC, n_pad, d), jnp.float32),
        mesh=mesh,
        scratch_types=[
            pltpu.VMEM_SHARED((n_pad, d), jnp.float32),
            pltpu.VMEM((nchunk, K), jnp.int32),
            pltpu.VMEM((K,), jnp.int32),
            pltpu.VMEM((K,), jnp.int32),
            pltpu.VMEM((K,), jnp.int32),
            pltpu.VMEM((K,), jnp.int32),
            pltpu.VMEM((K,), jnp.float32),
            pltpu.VMEM((K,), jnp.float32),
            pltpu.VMEM((K, d), jnp.float32),
            pltpu.VMEM((K, d), jnp.float32),
            pltpu.SemaphoreType.DMA,
            pltpu.SemaphoreType.DMA,
        ],
    )
    def spmm(emb_hbm, packed_hbm, val_hbm, part_hbm,
             acc_sh, packed_v, col0, col1, row0, row1, val0, val1,
             gbuf0, gbuf1, sem0, sem1):
        c = lax.axis_index("c")
        s = lax.axis_index("s")
        w = c * NS + s
        vbase = w * epw

        pltpu.async_copy(packed_hbm.at[w], packed_v, sem0)

        def zero_body(t, _):
            gbuf0[t // (d // L), pl.ds((t % (d // L)) * L, L)] = jnp.zeros(
                (L,), jnp.float32)
            return 0
        lax.fori_loop(0, K * (d // L), zero_body, 0)
        nfull, rem = rpt // K, rpt % K
        for i in range(nfull):
            pltpu.sync_copy(gbuf0, acc_sh.at[pl.ds(s * rpt + i * K, K)])
        if rem:
            pltpu.sync_copy(gbuf0.at[pl.ds(0, rem)],
                            acc_sh.at[pl.ds(s * rpt + nfull * K, rem)])
        pltpu.make_async_copy(packed_hbm.at[w], packed_v, sem0).wait()
        plsc.subcore_barrier()

        def unpack(ci, cbuf, rbuf):
            for g in range(K // L):
                p = packed_v[ci, pl.ds(g * L, L)]
                cbuf[pl.ds(g * L, L)] = lax.bitwise_and(p, mask)
                rbuf[pl.ds(g * L, L)] = lax.shift_right_logical(p, shift)

        def scale(buf, vbuf):
            def sb(g, _):
                k0 = g * L
                vv16 = vbuf[pl.ds(k0, L)]
                for e in range(L):
                    vvb = _lane_broadcast(vv16, e)
                    for j in range(d // L):
                        buf[k0 + e, pl.ds(j * L, L)] = (
                            buf[k0 + e, pl.ds(j * L, L)] * vvb)
                return 0
            lax.fori_loop(0, K // L, sb, 0)

        def gather(ci, buf, cbuf, vbuf, sem):
            pltpu.async_copy(emb_hbm.at[cbuf], buf, sem)
            pltpu.async_copy(val_hbm.at[pl.ds(vbase + ci * K, K)], vbuf, sem)

        def gather_wait(ci, buf, cbuf, vbuf, sem):
            pltpu.make_async_copy(emb_hbm.at[cbuf], buf, sem).wait()
            pltpu.make_async_copy(
                val_hbm.at[pl.ds(vbase + ci * K, K)], vbuf, sem).wait()

        unpack(0, col0, row0)
        gather(0, gbuf0, col0, val0, sem0)

        def chunk_pair(h, _):
            i0 = 2 * h

            @pl.when(i0 + 1 < nchunk)
            def _():
                unpack(i0 + 1, col1, row1)
                gather(i0 + 1, gbuf1, col1, val1, sem1)
            gather_wait(i0, gbuf0, col0, val0, sem0)
            scale(gbuf0, val0)
            pltpu.sync_copy(gbuf0, acc_sh.at[row0], add=True)

            @pl.when(i0 + 2 < nchunk)
            def _():
                unpack(i0 + 2, col0, row0)
                gather(i0 + 2, gbuf0, col0, val0, sem0)

            @pl.when(i0 + 1 < nchunk)
            def _():
                gather_wait(i0 + 1, gbuf1, col1, val1, sem1)
                scale(gbuf1, val1)
                pltpu.sync_copy(gbuf1, acc_sh.at[row1], add=True)
            return 0
        lax.fori_loop(0, (nchunk + 1) // 2, chunk_pair, 0)

        plsc.subcore_barrier()
        pltpu.sync_copy(acc_sh.at[pl.ds(s * rpt, rpt)],
                        part_hbm.at[c, pl.ds(s * rpt, rpt)])

    return spmm


def _add_block(a_ref, b_ref, o_ref):
    o_ref[...] = a_ref[0] + b_ref[0]


def _combine(partials, num_user, num_item, d, block):
    nbu, nbi = num_user // block, num_item // block
    assert nbu * block == num_user and nbi * block == num_item

    def call(nblk, row0):
        return pl.pallas_call(
            _add_block,
            grid=(nblk,),
            in_specs=[
                pl.BlockSpec((1, block, d), lambda i: (0, row0 + i, 0)),
                pl.BlockSpec((1, block, d), lambda i: (1, row0 + i, 0)),
            ],
            out_specs=pl.BlockSpec((block, d), lambda i: (i, 0)),
            out_shape=jax.ShapeDtypeStruct((nblk * block, d), jnp.float32),
        )(partials, partials)

    return call(nbu, 0), call(nbi, nbu)


def kernel(users_emb, items_emb, graph_row, graph_col, graph_val):
    num_user, d = users_emb.shape
    num_item = items_emb.shape[0]
    n_nodes = num_user + num_item
    n_edges = graph_row.shape[0]
    nw = NC * NS
    nchunk = n_edges // (nw * K)

    all_emb = jnp.concatenate([users_emb, items_emb], axis=0)
    shift = max(1, (n_nodes - 1).bit_length())
    packed = jnp.left_shift(graph_row, shift) | graph_col
    spmm = _spmm_partials(n_nodes, d, n_edges)
    partials = spmm(all_emb, packed.reshape(nw, nchunk, K), graph_val)
    return _combine(partials, num_user, num_item, d, 1000)


# speedup vs baseline: 12.8192x; 1.1254x over previous
import functools

import jax
import jax.numpy as jnp
from jax import lax
from jax.experimental import pallas as pl
from jax.experimental.pallas import tpu as pltpu
from jax.experimental.pallas import tpu_sc as plsc

NC = 2
NS = 16
L = 16
K = 80
NB = 3


def _lane_broadcast(vec, e):
    dnums = lax.GatherDimensionNumbers(
        offset_dims=(), collapsed_slice_dims=(0,), start_index_map=(0,))
    return lax.gather(vec, jnp.full((L, 1), e, jnp.int32), dnums, (1,),
                      mode=lax.GatherScatterMode.PROMISE_IN_BOUNDS)


def _spmm_partials(n_nodes, d, n_edges):
    shift = max(1, (n_nodes - 1).bit_length())
    mask = (1 << shift) - 1
    assert n_nodes * (1 << shift) <= 2**31
    nw = NC * NS
    epw = n_edges // nw
    nchunk = epw // K
    n_pad = -(-n_nodes // (NS * 8)) * (NS * 8)
    rpt = n_pad // NS
    assert epw * nw == n_edges and nchunk * K == epw
    assert d % L == 0 and K % L == 0

    mesh = plsc.VectorSubcoreMesh(core_axis_name="c", subcore_axis_name="s")

    @functools.partial(
        pl.kernel,
        out_type=jax.ShapeDtypeStruct((NC, n_pad, d), jnp.float32),
        mesh=mesh,
        scratch_types=[
            pltpu.VMEM_SHARED((n_pad, d), jnp.float32),
            pltpu.VMEM((nchunk, K), jnp.int32),
            pltpu.VMEM((NB, K), jnp.int32),
            pltpu.VMEM((NB, K), jnp.int32),
            pltpu.VMEM((NB, K), jnp.float32),
            pltpu.VMEM((NB, K, d), jnp.float32),
            pltpu.SemaphoreType.DMA,
            pltpu.SemaphoreType.DMA,
            pltpu.SemaphoreType.DMA,
            pltpu.SemaphoreType.DMA,
            pltpu.SemaphoreType.DMA,
            pltpu.SemaphoreType.DMA,
        ],
    )
    def spmm(emb_hbm, packed_hbm, val_hbm, part_hbm,
             acc_sh, packed_v, colb, rowb, valb, gbufb,
             gs0, gs1, gs2, ss0, ss1, ss2):
        gsem = (gs0, gs1, gs2)
        ssem = (ss0, ss1, ss2)
        c = lax.axis_index("c")
        s = lax.axis_index("s")
        w = c * NS + s
        vbase = w * epw

        pltpu.async_copy(packed_hbm.at[w], packed_v, gs0)

        def zero_body(t, _):
            gbufb[0, t // (d // L), pl.ds((t % (d // L)) * L, L)] = jnp.zeros(
                (L,), jnp.float32)
            return 0
        lax.fori_loop(0, K * (d // L), zero_body, 0)
        nfull, rem = rpt // K, rpt % K
        for i in range(nfull):
            pltpu.sync_copy(gbufb.at[0], acc_sh.at[pl.ds(s * rpt + i * K, K)])
        if rem:
            pltpu.sync_copy(gbufb.at[0, pl.ds(0, rem)],
                            acc_sh.at[pl.ds(s * rpt + nfull * K, rem)])
        pltpu.make_async_copy(packed_hbm.at[w], packed_v, gs0).wait()
        plsc.subcore_barrier()

        def unpack(ci, b):
            for g in range(K // L):
                p = packed_v[ci, pl.ds(g * L, L)]
                colb[b, pl.ds(g * L, L)] = lax.bitwise_and(p, mask)
                rowb[b, pl.ds(g * L, L)] = lax.shift_right_logical(p, shift)

        def scale(b):
            def sb(g, _):
                k0 = g * L
                vv16 = valb[b, pl.ds(k0, L)]
                for e in range(L):
                    vvb = _lane_broadcast(vv16, e)
                    for j in range(d // L):
                        gbufb[b, k0 + e, pl.ds(j * L, L)] = (
                            gbufb[b, k0 + e, pl.ds(j * L, L)] * vvb)
                return 0
            lax.fori_loop(0, K // L, sb, 0)

        def gather(ci, b):
            pltpu.async_copy(emb_hbm.at[colb.at[b]], gbufb.at[b], gsem[b])
            pltpu.async_copy(val_hbm.at[pl.ds(vbase + ci * K, K)],
                             valb.at[b], gsem[b])

        def gather_wait(ci, b):
            pltpu.make_async_copy(
                emb_hbm.at[colb.at[b]], gbufb.at[b], gsem[b]).wait()
            pltpu.make_async_copy(val_hbm.at[pl.ds(vbase + ci * K, K)],
                                  valb.at[b], gsem[b]).wait()

        def scatter(b):
            pltpu.async_copy(gbufb.at[b], acc_sh.at[rowb.at[b]], ssem[b],
                             add=True)

        def scatter_wait(b):
            pltpu.make_async_copy(gbufb.at[b], acc_sh.at[rowb.at[b]],
                                  ssem[b]).wait()

        unpack(0, 0)
        gather(0, 0)
        unpack(1, 1)
        gather(1, 1)

        def step(h, _):
            for t in range(NB):
                i = NB * h + t
                b = t
                bj = (t + NB - 1) % NB

                @pl.when(i < nchunk)
                def _():
                    gather_wait(i, b)
                    scale(b)
                    scatter(b)

                    @pl.when(i + NB - 1 < nchunk)
                    def _():
                        @pl.when(i >= 1)
                        def _():
                            scatter_wait(bj)
                        unpack(i + NB - 1, bj)
                        gather(i + NB - 1, bj)
            return 0
        lax.fori_loop(0, -(-nchunk // NB), step, 0)

        for b in range(NB):
            scatter_wait(b)
        plsc.subcore_barrier()
        pltpu.sync_copy(acc_sh.at[pl.ds(s * rpt, rpt)],
                        part_hbm.at[c, pl.ds(s * rpt, rpt)])

    return spmm


def _add_block(a_ref, b_ref, o_ref):
    o_ref[...] = a_ref[0] + b_ref[0]


def _combine(partials, num_user, num_item, d, block):
    nbu, nbi = num_user // block, num_item // block
    assert nbu * block == num_user and nbi * block == num_item

    def call(nblk, row0):
        return pl.pallas_call(
            _add_block,
            grid=(nblk,),
            in_specs=[
                pl.BlockSpec((1, block, d), lambda i: (0, row0 + i, 0)),
                pl.BlockSpec((1, block, d), lambda i: (1, row0 + i, 0)),
            ],
            out_specs=pl.BlockSpec((block, d), lambda i: (i, 0)),
            out_shape=jax.ShapeDtypeStruct((nblk * block, d), jnp.float32),
        )(partials, partials)

    return call(nbu, 0), call(nbi, nbu)


def kernel(users_emb, items_emb, graph_row, graph_col, graph_val):
    num_user, d = users_emb.shape
    num_item = items_emb.shape[0]
    n_nodes = num_user + num_item
    n_edges = graph_row.shape[0]
    nw = NC * NS
    nchunk = n_edges // (nw * K)

    all_emb = jnp.concatenate([users_emb, items_emb], axis=0)
    shift = max(1, (n_nodes - 1).bit_length())
    packed = jnp.left_shift(graph_row, shift) | graph_col
    spmm = _spmm_partials(n_nodes, d, n_edges)
    partials = spmm(all_emb, packed.reshape(nw, nchunk, K), graph_val)
    return _combine(partials, num_user, num_item, d, 1000)


# speedup vs baseline: 12.8822x; 1.0049x over previous
import functools

import jax
import jax.numpy as jnp
from jax import lax
from jax.experimental import pallas as pl
from jax.experimental.pallas import tpu as pltpu
from jax.experimental.pallas import tpu_sc as plsc

NC = 2
NS = 16
L = 16
K = 80
NB = 3


def _lane_broadcast(vec, e):
    dnums = lax.GatherDimensionNumbers(
        offset_dims=(), collapsed_slice_dims=(0,), start_index_map=(0,))
    return lax.gather(vec, jnp.full((L, 1), e, jnp.int32), dnums, (1,),
                      mode=lax.GatherScatterMode.PROMISE_IN_BOUNDS)


def _spmm_partials(n_nodes, d, n_edges):
    shift = max(1, (n_nodes - 1).bit_length())
    mask = (1 << shift) - 1
    assert n_nodes * (1 << shift) <= 2**31
    nw = NC * NS
    epw = n_edges // nw
    nchunk = epw // K
    n_pad = -(-n_nodes // (NS * 8)) * (NS * 8)
    rpt = n_pad // NS
    assert epw * nw == n_edges and nchunk * K == epw
    assert d % L == 0 and K % L == 0
    assert nchunk >= NB

    mesh = plsc.VectorSubcoreMesh(core_axis_name="c", subcore_axis_name="s")

    @functools.partial(
        pl.kernel,
        out_type=jax.ShapeDtypeStruct((NC, n_pad, d), jnp.float32),
        mesh=mesh,
        scratch_types=[
            pltpu.VMEM_SHARED((n_pad, d), jnp.float32),
            pltpu.VMEM((NB, K), jnp.int32),
            pltpu.VMEM((NB, K), jnp.int32),
            pltpu.VMEM((NB, K), jnp.int32),
            pltpu.VMEM((NB, K), jnp.float32),
            pltpu.VMEM((NB, K, d), jnp.float32),
            pltpu.SemaphoreType.DMA,
            pltpu.SemaphoreType.DMA,
            pltpu.SemaphoreType.DMA,
            pltpu.SemaphoreType.DMA,
            pltpu.SemaphoreType.DMA,
            pltpu.SemaphoreType.DMA,
            pltpu.SemaphoreType.DMA,
            pltpu.SemaphoreType.DMA,
            pltpu.SemaphoreType.DMA,
        ],
    )
    def spmm(emb_hbm, packed_hbm, val_hbm, part_hbm,
             acc_sh, pbuf, colb, rowb, valb, gbufb,
             gs0, gs1, gs2, ss0, ss1, ss2, ps0, ps1, ps2):
        gsem = (gs0, gs1, gs2)
        ssem = (ss0, ss1, ss2)
        psem = (ps0, ps1, ps2)
        c = lax.axis_index("c")
        s = lax.axis_index("s")
        w = c * NS + s
        vbase = w * epw

        def pfetch(ci, b):
            pltpu.async_copy(packed_hbm.at[w, ci], pbuf.at[b], psem[b])

        def pfetch_wait(ci, b):
            pltpu.make_async_copy(
                packed_hbm.at[w, ci], pbuf.at[b], psem[b]).wait()

        def unpack(b):
            for g in range(K // L):
                p = pbuf[b, pl.ds(g * L, L)]
                colb[b, pl.ds(g * L, L)] = lax.bitwise_and(p, mask)
                rowb[b, pl.ds(g * L, L)] = lax.shift_right_logical(p, shift)

        def scale(b):
            def sb(g, _):
                k0 = g * L
                vv16 = valb[b, pl.ds(k0, L)]
                for e in range(L):
                    vvb = _lane_broadcast(vv16, e)
                    for j in range(d // L):
                        gbufb[b, k0 + e, pl.ds(j * L, L)] = (
                            gbufb[b, k0 + e, pl.ds(j * L, L)] * vvb)
                return 0
            lax.fori_loop(0, K // L, sb, 0)

        def gather(ci, b):
            pltpu.async_copy(emb_hbm.at[colb.at[b]], gbufb.at[b], gsem[b])
            pltpu.async_copy(val_hbm.at[pl.ds(vbase + ci * K, K)],
                             valb.at[b], gsem[b])

        def gather_wait(ci, b):
            pltpu.make_async_copy(
                emb_hbm.at[colb.at[b]], gbufb.at[b], gsem[b]).wait()
            pltpu.make_async_copy(val_hbm.at[pl.ds(vbase + ci * K, K)],
                                  valb.at[b], gsem[b]).wait()

        def scatter(b):
            pltpu.async_copy(gbufb.at[b], acc_sh.at[rowb.at[b]], ssem[b],
                             add=True)

        def scatter_wait(b):
            pltpu.make_async_copy(gbufb.at[b], acc_sh.at[rowb.at[b]],
                                  ssem[b]).wait()

        for b0 in range(NB - 1):
            pfetch(b0, b0)
        for b0 in range(NB - 1):
            pfetch_wait(b0, b0)
            unpack(b0)
            gather(b0, b0)
        pfetch(NB - 1, NB - 1)

        zb = NB - 1

        def zero_body(t, _):
            gbufb[zb, t // (d // L), pl.ds((t % (d // L)) * L, L)] = jnp.zeros(
                (L,), jnp.float32)
            return 0
        lax.fori_loop(0, K * (d // L), zero_body, 0)
        nfull, rem = rpt // K, rpt % K
        for i in range(nfull):
            pltpu.sync_copy(gbufb.at[zb], acc_sh.at[pl.ds(s * rpt + i * K, K)])
        if rem:
            pltpu.sync_copy(gbufb.at[zb, pl.ds(0, rem)],
                            acc_sh.at[pl.ds(s * rpt + nfull * K, rem)])
        plsc.subcore_barrier()

        def step(h, _):
            for t in range(NB):
                i = NB * h + t
                b = t
                bj = (t + NB - 1) % NB

                @pl.when(i < nchunk)
                def _():
                    gather_wait(i, b)
                    scale(b)
                    scatter(b)

                    @pl.when(i + NB - 1 < nchunk)
                    def _():
                        @pl.when(i >= 1)
                        def _():
                            scatter_wait(bj)
                        pfetch_wait(i + NB - 1, bj)
                        unpack(bj)
                        gather(i + NB - 1, bj)

                        @pl.when(i + NB < nchunk)
                        def _():
                            pfetch(i + NB, (bj + 1) % NB)
            return 0
        lax.fori_loop(0, -(-nchunk // NB), step, 0)

        for b in range(NB):
            scatter_wait(b)
        plsc.subcore_barrier()
        pltpu.sync_copy(acc_sh.at[pl.ds(s * rpt, rpt)],
                        part_hbm.at[c, pl.ds(s * rpt, rpt)])

    return spmm


def _add_block(a_ref, b_ref, o_ref):
    o_ref[...] = a_ref[0] + b_ref[0]


def _combine(partials, num_user, num_item, d, block):
    nbu, nbi = num_user // block, num_item // block
    assert nbu * block == num_user and nbi * block == num_item

    def call(nblk, row0):
        return pl.pallas_call(
            _add_block,
            grid=(nblk,),
            in_specs=[
                pl.BlockSpec((1, block, d), lambda i: (0, row0 + i, 0)),
                pl.BlockSpec((1, block, d), lambda i: (1, row0 + i, 0)),
            ],
            out_specs=pl.BlockSpec((block, d), lambda i: (i, 0)),
            out_shape=jax.ShapeDtypeStruct((nblk * block, d), jnp.float32),
        )(partials, partials)

    return call(nbu, 0), call(nbi, nbu)


def kernel(users_emb, items_emb, graph_row, graph_col, graph_val):
    num_user, d = users_emb.shape
    num_item = items_emb.shape[0]
    n_nodes = num_user + num_item
    n_edges = graph_row.shape[0]
    nw = NC * NS
    nchunk = n_edges // (nw * K)

    all_emb = jnp.concatenate([users_emb, items_emb], axis=0)
    shift = max(1, (n_nodes - 1).bit_length())
    packed = jnp.left_shift(graph_row, shift) | graph_col
    spmm = _spmm_partials(n_nodes, d, n_edges)
    partials = spmm(all_emb, packed.reshape(nw, nchunk, K), graph_val)
    return _combine(partials, num_user, num_item, d, 1000)


# speedup vs baseline: 12.8934x; 1.0009x over previous
import functools

import jax
import jax.numpy as jnp
from jax import lax
from jax.experimental import pallas as pl
from jax.experimental.pallas import tpu as pltpu
from jax.experimental.pallas import tpu_sc as plsc

NC = 2
NS = 16
L = 16
K = 80
NB = 3


def _lane_broadcast(vec, e):
    dnums = lax.GatherDimensionNumbers(
        offset_dims=(), collapsed_slice_dims=(0,), start_index_map=(0,))
    return lax.gather(vec, jnp.full((L, 1), e, jnp.int32), dnums, (1,),
                      mode=lax.GatherScatterMode.PROMISE_IN_BOUNDS)


def _spmm_partials(n_nodes, d, n_edges):
    shift = max(1, (n_nodes - 1).bit_length())
    mask = (1 << shift) - 1
    assert n_nodes * (1 << shift) <= 2**31
    nw = NC * NS
    epw = n_edges // nw
    nchunk = epw // K
    n_pad = -(-n_nodes // (NS * 8)) * (NS * 8)
    rpt = n_pad // NS
    assert epw * nw == n_edges and nchunk * K == epw
    assert d % L == 0 and K % L == 0
    assert nchunk >= NB

    mesh = plsc.VectorSubcoreMesh(core_axis_name="c", subcore_axis_name="s")

    @functools.partial(
        pl.kernel,
        out_type=jax.ShapeDtypeStruct((NC, n_pad, d), jnp.float32),
        mesh=mesh,
        scratch_types=[
            pltpu.VMEM_SHARED((n_pad, d), jnp.float32),
            pltpu.VMEM((NB, K), jnp.int32),
            pltpu.VMEM((NB, K), jnp.int32),
            pltpu.VMEM((NB, K), jnp.int32),
            pltpu.VMEM((epw,), jnp.float32),
            pltpu.VMEM((NB, K, d), jnp.float32),
            pltpu.SemaphoreType.DMA,
            pltpu.SemaphoreType.DMA,
            pltpu.SemaphoreType.DMA,
            pltpu.SemaphoreType.DMA,
            pltpu.SemaphoreType.DMA,
            pltpu.SemaphoreType.DMA,
            pltpu.SemaphoreType.DMA,
            pltpu.SemaphoreType.DMA,
            pltpu.SemaphoreType.DMA,
            pltpu.SemaphoreType.DMA,
        ],
    )
    def spmm(emb_hbm, packed_hbm, val_hbm, part_hbm,
             acc_sh, pbuf, colb, rowb, vall, gbufb,
             gs0, gs1, gs2, ss0, ss1, ss2, ps0, ps1, ps2, vs0):
        gsem = (gs0, gs1, gs2)
        ssem = (ss0, ss1, ss2)
        psem = (ps0, ps1, ps2)
        c = lax.axis_index("c")
        s = lax.axis_index("s")
        w = c * NS + s
        vbase = w * epw

        def pfetch(ci, b):
            pltpu.async_copy(packed_hbm.at[w, ci], pbuf.at[b], psem[b])

        def pfetch_wait(ci, b):
            pltpu.make_async_copy(
                packed_hbm.at[w, ci], pbuf.at[b], psem[b]).wait()

        def unpack(b):
            for g in range(K // L):
                p = pbuf[b, pl.ds(g * L, L)]
                colb[b, pl.ds(g * L, L)] = lax.bitwise_and(p, mask)
                rowb[b, pl.ds(g * L, L)] = lax.shift_right_logical(p, shift)

        def scale(ci, b):
            def sb(g, _):
                k0 = g * L
                vv16 = vall[pl.ds(ci * K + k0, L)]
                for e in range(L):
                    vvb = _lane_broadcast(vv16, e)
                    for j in range(d // L):
                        gbufb[b, k0 + e, pl.ds(j * L, L)] = (
                            gbufb[b, k0 + e, pl.ds(j * L, L)] * vvb)
                return 0
            lax.fori_loop(0, K // L, sb, 0)

        def gather(ci, b):
            pltpu.async_copy(emb_hbm.at[colb.at[b]], gbufb.at[b], gsem[b])

        def gather_wait(ci, b):
            pltpu.make_async_copy(
                emb_hbm.at[colb.at[b]], gbufb.at[b], gsem[b]).wait()

        def scatter(b):
            pltpu.async_copy(gbufb.at[b], acc_sh.at[rowb.at[b]], ssem[b],
                             add=True)

        def scatter_wait(b):
            pltpu.make_async_copy(gbufb.at[b], acc_sh.at[rowb.at[b]],
                                  ssem[b]).wait()

        pltpu.async_copy(val_hbm.at[pl.ds(vbase, epw)], vall, vs0)
        for b0 in range(NB - 1):
            pfetch(b0, b0)
        for b0 in range(NB - 1):
            pfetch_wait(b0, b0)
            unpack(b0)
            gather(b0, b0)
        pfetch(NB - 1, NB - 1)

        zb = NB - 1

        def zero_body(t, _):
            gbufb[zb, t // (d // L), pl.ds((t % (d // L)) * L, L)] = jnp.zeros(
                (L,), jnp.float32)
            return 0
        lax.fori_loop(0, K * (d // L), zero_body, 0)
        nfull, rem = rpt // K, rpt % K
        for i in range(nfull):
            pltpu.sync_copy(gbufb.at[zb], acc_sh.at[pl.ds(s * rpt + i * K, K)])
        if rem:
            pltpu.sync_copy(gbufb.at[zb, pl.ds(0, rem)],
                            acc_sh.at[pl.ds(s * rpt + nfull * K, rem)])
        pltpu.make_async_copy(val_hbm.at[pl.ds(vbase, epw)], vall,
                              vs0).wait()
        plsc.subcore_barrier()

        def step(h, _):
            for t in range(NB):
                i = NB * h + t
                b = t
                bj = (t + NB - 1) % NB

                @pl.when(i < nchunk)
                def _():
                    gather_wait(i, b)
                    scale(i, b)
                    scatter(b)

                    @pl.when(i + NB - 1 < nchunk)
                    def _():
                        @pl.when(i >= 1)
                        def _():
                            scatter_wait(bj)
                        pfetch_wait(i + NB - 1, bj)
                        unpack(bj)
                        gather(i + NB - 1, bj)

                        @pl.when(i + NB < nchunk)
                        def _():
                            pfetch(i + NB, (bj + 1) % NB)
            return 0
        lax.fori_loop(0, -(-nchunk // NB), step, 0)

        for b in range(NB):
            scatter_wait(b)
        plsc.subcore_barrier()
        pltpu.sync_copy(acc_sh.at[pl.ds(s * rpt, rpt)],
                        part_hbm.at[c, pl.ds(s * rpt, rpt)])

    return spmm


def _add_block(a_ref, b_ref, o_ref):
    o_ref[...] = a_ref[0] + b_ref[0]


def _combine(partials, num_user, num_item, d, block):
    nbu, nbi = num_user // block, num_item // block
    assert nbu * block == num_user and nbi * block == num_item

    def call(nblk, row0):
        return pl.pallas_call(
            _add_block,
            grid=(nblk,),
            in_specs=[
                pl.BlockSpec((1, block, d), lambda i: (0, row0 + i, 0)),
                pl.BlockSpec((1, block, d), lambda i: (1, row0 + i, 0)),
            ],
            out_specs=pl.BlockSpec((block, d), lambda i: (i, 0)),
            out_shape=jax.ShapeDtypeStruct((nblk * block, d), jnp.float32),
        )(partials, partials)

    return call(nbu, 0), call(nbi, nbu)


def kernel(users_emb, items_emb, graph_row, graph_col, graph_val):
    num_user, d = users_emb.shape
    num_item = items_emb.shape[0]
    n_nodes = num_user + num_item
    n_edges = graph_row.shape[0]
    nw = NC * NS
    nchunk = n_edges // (nw * K)

    all_emb = jnp.concatenate([users_emb, items_emb], axis=0)
    shift = max(1, (n_nodes - 1).bit_length())
    packed = jnp.left_shift(graph_row, shift) | graph_col
    spmm = _spmm_partials(n_nodes, d, n_edges)
    partials = spmm(all_emb, packed.reshape(nw, nchunk, K), graph_val)
    return _combine(partials, num_user, num_item, d, 1000)


# speedup vs baseline: 13.1674x; 1.0212x over previous
import functools

import jax
import jax.numpy as jnp
from jax import lax
from jax.experimental import pallas as pl
from jax.experimental.pallas import tpu as pltpu
from jax.experimental.pallas import tpu_sc as plsc

NC = 2
NS = 16
L = 16
K = 80
NB = 4


def _lane_broadcast(vec, e):
    dnums = lax.GatherDimensionNumbers(
        offset_dims=(), collapsed_slice_dims=(0,), start_index_map=(0,))
    return lax.gather(vec, jnp.full((L, 1), e, jnp.int32), dnums, (1,),
                      mode=lax.GatherScatterMode.PROMISE_IN_BOUNDS)


def _spmm_partials(n_nodes, d, n_edges):
    shift = max(1, (n_nodes - 1).bit_length())
    mask = (1 << shift) - 1
    assert n_nodes * (1 << shift) <= 2**31
    nw = NC * NS
    epw = n_edges // nw
    nchunk = epw // K
    n_pad = -(-n_nodes // (NS * 8)) * (NS * 8)
    rpt = n_pad // NS
    assert epw * nw == n_edges and nchunk * K == epw
    assert d % L == 0 and K % L == 0
    assert nchunk >= NB

    mesh = plsc.VectorSubcoreMesh(core_axis_name="c", subcore_axis_name="s")

    @functools.partial(
        pl.kernel,
        out_type=jax.ShapeDtypeStruct((NC, n_pad, d), jnp.float32),
        mesh=mesh,
        scratch_types=[
            pltpu.VMEM_SHARED((n_pad, d), jnp.float32),
            pltpu.VMEM((NB, K), jnp.int32),
            pltpu.VMEM((NB, K), jnp.int32),
            pltpu.VMEM((NB, K), jnp.int32),
            pltpu.VMEM((NB, K), jnp.float32),
            pltpu.VMEM((NB, K, d), jnp.float32),
            pltpu.SemaphoreType.DMA,
            pltpu.SemaphoreType.DMA,
            pltpu.SemaphoreType.DMA,
            pltpu.SemaphoreType.DMA,
            pltpu.SemaphoreType.DMA,
            pltpu.SemaphoreType.DMA,
            pltpu.SemaphoreType.DMA,
            pltpu.SemaphoreType.DMA,
            pltpu.SemaphoreType.DMA,
            pltpu.SemaphoreType.DMA,
            pltpu.SemaphoreType.DMA,
            pltpu.SemaphoreType.DMA,
            pltpu.SemaphoreType.DMA,
            pltpu.SemaphoreType.DMA,
            pltpu.SemaphoreType.DMA,
            pltpu.SemaphoreType.DMA,
        ],
    )
    def spmm(emb_hbm, packed_hbm, val_hbm, part_hbm,
             acc_sh, pbuf, colb, rowb, vbuf, gbufb,
             gs0, gs1, gs2, gs3, ss0, ss1, ss2, ss3,
             ps0, ps1, ps2, ps3, vs0, vs1, vs2, vs3):
        gsem = (gs0, gs1, gs2, gs3)
        ssem = (ss0, ss1, ss2, ss3)
        psem = (ps0, ps1, ps2, ps3)
        vsem = (vs0, vs1, vs2, vs3)
        c = lax.axis_index("c")
        s = lax.axis_index("s")
        w = c * NS + s

        def pfetch(ci, b):
            pltpu.async_copy(packed_hbm.at[w, ci], pbuf.at[b], psem[b])
            pltpu.async_copy(val_hbm.at[w, ci], vbuf.at[b], vsem[b])

        def pfetch_wait(ci, b):
            pltpu.make_async_copy(
                packed_hbm.at[w, ci], pbuf.at[b], psem[b]).wait()
            pltpu.make_async_copy(
                val_hbm.at[w, ci], vbuf.at[b], vsem[b]).wait()

        def unpack(b):
            for g in range(K // L):
                p = pbuf[b, pl.ds(g * L, L)]
                colb[b, pl.ds(g * L, L)] = lax.bitwise_and(p, mask)
                rowb[b, pl.ds(g * L, L)] = lax.shift_right_logical(p, shift)

        def scale(ci, b):
            def sb(g, _):
                k0 = g * L
                vv16 = vbuf[b, pl.ds(k0, L)]
                for e in range(L):
                    vvb = _lane_broadcast(vv16, e)
                    for j in range(d // L):
                        gbufb[b, k0 + e, pl.ds(j * L, L)] = (
                            gbufb[b, k0 + e, pl.ds(j * L, L)] * vvb)
                return 0
            lax.fori_loop(0, K // L, sb, 0)

        def gather(ci, b):
            pltpu.async_copy(emb_hbm.at[colb.at[b]], gbufb.at[b], gsem[b])

        def gather_wait(ci, b):
            pltpu.make_async_copy(
                emb_hbm.at[colb.at[b]], gbufb.at[b], gsem[b]).wait()

        def scatter(b):
            pltpu.async_copy(gbufb.at[b], acc_sh.at[rowb.at[b]], ssem[b],
                             add=True)

        def scatter_wait(b):
            pltpu.make_async_copy(gbufb.at[b], acc_sh.at[rowb.at[b]],
                                  ssem[b]).wait()

        for b0 in range(NB - 1):
            pfetch(b0, b0)
        for b0 in range(NB - 1):
            pfetch_wait(b0, b0)
            unpack(b0)
            gather(b0, b0)
        pfetch(NB - 1, NB - 1)

        zb = NB - 1

        def zero_body(t, _):
            gbufb[zb, t // (d // L), pl.ds((t % (d // L)) * L, L)] = jnp.zeros(
                (L,), jnp.float32)
            return 0
        lax.fori_loop(0, K * (d // L), zero_body, 0)
        nfull, rem = rpt // K, rpt % K
        for i in range(nfull):
            pltpu.sync_copy(gbufb.at[zb], acc_sh.at[pl.ds(s * rpt + i * K, K)])
        if rem:
            pltpu.sync_copy(gbufb.at[zb, pl.ds(0, rem)],
                            acc_sh.at[pl.ds(s * rpt + nfull * K, rem)])
        plsc.subcore_barrier()

        def step(h, _):
            for t in range(NB):
                i = NB * h + t
                b = t
                bj = (t + NB - 1) % NB

                @pl.when(i < nchunk)
                def _():
                    gather_wait(i, b)
                    scale(i, b)
                    scatter(b)

                    @pl.when(i + NB - 1 < nchunk)
                    def _():
                        @pl.when(i >= 1)
                        def _():
                            scatter_wait(bj)
                        pfetch_wait(i + NB - 1, bj)
                        unpack(bj)
                        gather(i + NB - 1, bj)

                        @pl.when(i + NB < nchunk)
                        def _():
                            pfetch(i + NB, (bj + 1) % NB)
            return 0
        lax.fori_loop(0, -(-nchunk // NB), step, 0)

        for b in range(NB):
            scatter_wait(b)
        plsc.subcore_barrier()
        pltpu.sync_copy(acc_sh.at[pl.ds(s * rpt, rpt)],
                        part_hbm.at[c, pl.ds(s * rpt, rpt)])

    return spmm


def _add_block(a_ref, b_ref, o_ref):
    o_ref[...] = a_ref[0] + b_ref[0]


def _combine(partials, num_user, num_item, d, block):
    nbu, nbi = num_user // block, num_item // block
    assert nbu * block == num_user and nbi * block == num_item

    def call(nblk, row0):
        return pl.pallas_call(
            _add_block,
            grid=(nblk,),
            in_specs=[
                pl.BlockSpec((1, block, d), lambda i: (0, row0 + i, 0)),
                pl.BlockSpec((1, block, d), lambda i: (1, row0 + i, 0)),
            ],
            out_specs=pl.BlockSpec((block, d), lambda i: (i, 0)),
            out_shape=jax.ShapeDtypeStruct((nblk * block, d), jnp.float32),
        )(partials, partials)

    return call(nbu, 0), call(nbi, nbu)


def kernel(users_emb, items_emb, graph_row, graph_col, graph_val):
    num_user, d = users_emb.shape
    num_item = items_emb.shape[0]
    n_nodes = num_user + num_item
    n_edges = graph_row.shape[0]
    nw = NC * NS
    nchunk = n_edges // (nw * K)

    all_emb = jnp.concatenate([users_emb, items_emb], axis=0)
    shift = max(1, (n_nodes - 1).bit_length())
    packed = jnp.left_shift(graph_row, shift) | graph_col
    spmm = _spmm_partials(n_nodes, d, n_edges)
    partials = spmm(all_emb, packed.reshape(nw, nchunk, K),
                    graph_val.reshape(nw, nchunk, K))
    return _combine(partials, num_user, num_item, d, 1000)
